# baseline (speedup 1.0000x reference)
import jax
import jax.numpy as jnp
from jax.experimental import pallas as pl
from jax.experimental.pallas import tpu as pltpu

N_E = 8192
E_DIM = 64
BM = 1024
W_BOUNDS = (0, 4096, 8192)


def _vq_body(zf_ref, e_ref, out_ref, en_ref):
    e = e_ref[...]

    @pl.when(pl.program_id(0) == 0)
    def _():
        en_ref[...] = jnp.sum(e ** 2, axis=1)

    zb = zf_ref[...]
    zn = jnp.sum(zb ** 2, axis=1, keepdims=True)

    zb_m2 = zb * -2.0
    en = en_ref[...]

    run_v = jnp.full((BM,), jnp.inf, dtype=jnp.float32)
    run_i = jnp.zeros((BM,), dtype=jnp.int32)
    NL = 128
    for lo, hi in zip(W_BOUNDS[:-1], W_BOUNDS[1:]):
        dot_m2 = jax.lax.dot_general(
            zb_m2, e[lo:hi, :], (((1,), (1,)), ((), ())),
            preferred_element_type=jnp.float32,
        )
        def dgroup(g):
            c0 = lo + g * NL
            return (zn + en[c0:c0 + NL][None, :]) + dot_m2[:, g * NL:(g + 1) * NL]
        minv = dgroup(0)
        gidx = jnp.zeros((BM, NL), dtype=jnp.int32)
        for g in range(1, (hi - lo) // NL):
            blk = dgroup(g)
            pred = blk < minv
            minv = jnp.where(pred, blk, minv)
            gidx = jnp.where(pred, jnp.int32(g), gidx)
        wmin = jnp.min(minv, axis=1)
        lane = jax.lax.broadcasted_iota(jnp.int32, (BM, NL), 1)
        colidx = gidx * NL + lane
        cand = jnp.where(minv == wmin[:, None], colidx, jnp.int32(N_E))
        widx = jnp.min(cand, axis=1) + lo
        take = wmin < run_v
        run_i = jnp.where(take, widx, run_i)
        run_v = jnp.where(take, wmin, run_v)
        run_v = run_v.astype(jnp.bfloat16).astype(jnp.float32)
    out_ref[...] = run_i


@jax.jit
def kernel(z, embedding_weight):
    zf = jnp.transpose(z, (0, 2, 3, 1)).reshape(-1, E_DIM)
    m = zf.shape[0]
    out = pl.pallas_call(
        _vq_body,
        grid=(m // BM,),
        in_specs=[
            pl.BlockSpec((BM, E_DIM), lambda i: (i, 0)),
            pl.BlockSpec((N_E, E_DIM), lambda i: (0, 0)),
        ],
        out_specs=pl.BlockSpec((BM,), lambda i: (i,)),
        out_shape=jax.ShapeDtypeStruct((m,), jnp.int32),
        scratch_shapes=[pltpu.VMEM((N_E,), jnp.float32)],
    )(zf, embedding_weight)
    return out


# speedup vs baseline: 1.0306x; 1.0306x over previous
import jax
import jax.numpy as jnp
from jax.experimental import pallas as pl
from jax.experimental.pallas import tpu as pltpu

N_E = 8192
E_DIM = 64
BM = 2048
W_BOUNDS = (0, 4096, 8192)


def _vq_body(zf_ref, e_ref, out_ref, en_ref):
    e = e_ref[...]

    @pl.when(pl.program_id(0) == 0)
    def _():
        en_ref[...] = jnp.sum(e ** 2, axis=1)

    zb = zf_ref[...]
    zn = jnp.sum(zb ** 2, axis=1, keepdims=True)

    zb_m2 = zb * -2.0
    en = en_ref[...]

    run_v = jnp.full((BM,), jnp.inf, dtype=jnp.float32)
    run_i = jnp.zeros((BM,), dtype=jnp.int32)
    NL = 128
    for lo, hi in zip(W_BOUNDS[:-1], W_BOUNDS[1:]):
        dot_m2 = jax.lax.dot_general(
            zb_m2, e[lo:hi, :], (((1,), (1,)), ((), ())),
            preferred_element_type=jnp.float32,
        )
        def dgroup(g):
            c0 = lo + g * NL
            return (zn + en[c0:c0 + NL][None, :]) + dot_m2[:, g * NL:(g + 1) * NL]
        minv = dgroup(0)
        gidx = jnp.zeros((BM, NL), dtype=jnp.int32)
        for g in range(1, (hi - lo) // NL):
            blk = dgroup(g)
            pred = blk < minv
            minv = jnp.where(pred, blk, minv)
            gidx = jnp.where(pred, jnp.int32(g), gidx)
        wmin = jnp.min(minv, axis=1)
        lane = jax.lax.broadcasted_iota(jnp.int32, (BM, NL), 1)
        colidx = gidx * NL + lane
        cand = jnp.where(minv == wmin[:, None], colidx, jnp.int32(N_E))
        widx = jnp.min(cand, axis=1) + lo
        take = wmin < run_v
        run_i = jnp.where(take, widx, run_i)
        run_v = jnp.where(take, wmin, run_v)
        run_v = run_v.astype(jnp.bfloat16).astype(jnp.float32)
    out_ref[...] = run_i


@jax.jit
def kernel(z, embedding_weight):
    zf = jnp.transpose(z, (0, 2, 3, 1)).reshape(-1, E_DIM)
    m = zf.shape[0]
    out = pl.pallas_call(
        _vq_body,
        grid=(m // BM,),
        in_specs=[
            pl.BlockSpec((BM, E_DIM), lambda i: (i, 0)),
            pl.BlockSpec((N_E, E_DIM), lambda i: (0, 0)),
        ],
        out_specs=pl.BlockSpec((BM,), lambda i: (i,)),
        out_shape=jax.ShapeDtypeStruct((m,), jnp.int32),
        scratch_shapes=[pltpu.VMEM((N_E,), jnp.float32)],
    )(zf, embedding_weight)
    return out


# speedup vs baseline: 1.0980x; 1.0654x over previous
import jax
import jax.numpy as jnp
from jax.experimental import pallas as pl
from jax.experimental.pallas import tpu as pltpu

N_E = 8192
E_DIM = 64
BM = 2048
W_BOUNDS = (0, 4096, 8192)


def _vq_body(zf_ref, e_ref, out_ref):
    e = e_ref[...]
    en = jnp.sum(e ** 2, axis=1)

    zb = zf_ref[...]
    zn = jnp.sum(zb ** 2, axis=1, keepdims=True)

    zb_m2 = zb * -2.0

    run_v = jnp.full((BM,), jnp.inf, dtype=jnp.float32)
    run_i = jnp.zeros((BM,), dtype=jnp.int32)
    NL = 128
    for lo, hi in zip(W_BOUNDS[:-1], W_BOUNDS[1:]):
        dot_m2 = jax.lax.dot_general(
            zb_m2, e[lo:hi, :], (((1,), (1,)), ((), ())),
            preferred_element_type=jnp.float32,
        )
        def dgroup(g):
            c0 = lo + g * NL
            return (zn + en[c0:c0 + NL][None, :]) + dot_m2[:, g * NL:(g + 1) * NL]
        minv = dgroup(0)
        gidx = jnp.zeros((BM, NL), dtype=jnp.int32)
        for g in range(1, (hi - lo) // NL):
            blk = dgroup(g)
            pred = blk < minv
            minv = jnp.where(pred, blk, minv)
            gidx = jnp.where(pred, jnp.int32(g), gidx)
        wmin = jnp.min(minv, axis=1)
        lane = jax.lax.broadcasted_iota(jnp.int32, (BM, NL), 1)
        colidx = gidx * NL + lane
        cand = jnp.where(minv == wmin[:, None], colidx, jnp.int32(N_E))
        widx = jnp.min(cand, axis=1) + lo
        take = wmin < run_v
        run_i = jnp.where(take, widx, run_i)
        run_v = jnp.where(take, wmin, run_v)
        run_v = run_v.astype(jnp.bfloat16).astype(jnp.float32)
    out_ref[...] = run_i


@jax.jit
def kernel(z, embedding_weight):
    zf = jnp.transpose(z, (0, 2, 3, 1)).reshape(-1, E_DIM)
    m = zf.shape[0]
    out = pl.pallas_call(
        _vq_body,
        grid=(m // BM,),
        in_specs=[
            pl.BlockSpec((BM, E_DIM), lambda i: (i, 0)),
            pl.BlockSpec((N_E, E_DIM), lambda i: (0, 0)),
        ],
        out_specs=pl.BlockSpec((BM,), lambda i: (i,)),
        out_shape=jax.ShapeDtypeStruct((m,), jnp.int32),
        compiler_params=pltpu.CompilerParams(
            dimension_semantics=("parallel",),
        ),
    )(zf, embedding_weight)
    return out
